# baseline (device time: 1504867 ns/iter reference)
import jax
import jax.numpy as jnp
from jax import lax
from jax.experimental import pallas as pl
from jax.experimental.pallas import tpu as pltpu

NDEV = 32
M = 8192
KSH = 256
N = 4096
NH = N // 2
NSUB = 4
HQ = NH // NSUB
CH = M // NDEV

MESH = pl.DeviceIdType.MESH

_PLANE_ORDER = [(0, 0), (1, 0), (1, 1), (0, 1),
                (0, 2), (1, 2), (1, 3), (0, 3)]
_LOGICAL_OF_COORD = {}
for _z in range(4):
    for _x, _y in _PLANE_ORDER:
        _LOGICAL_OF_COORD[(_x, _y, _z)] = len(_LOGICAL_OF_COORD)

_PATH_YZ = [(0, 0), (1, 0), (2, 0), (3, 0), (3, 1), (2, 1), (1, 1), (0, 1),
            (0, 2), (1, 2), (2, 2), (3, 2), (3, 3), (2, 3), (1, 3), (0, 3)]
_RING_COORDS = ([(0, y, z) for y, z in _PATH_YZ]
                + [(1, y, z) for y, z in reversed(_PATH_YZ)])
RING_LOGICAL = [_LOGICAL_OF_COORD[c] for c in _RING_COORDS]
POS_OF_LOGICAL = [0] * NDEV
for _p, _l in enumerate(RING_LOGICAL):
    POS_OF_LOGICAL[_l] = _p


def kernel(x, w_mat):
    def body(scal_ref, x_ref, w_ref, out_ref,
             acc_cw, acc_ccw, tmp_cw, tmp_ccw, comm_cw, comm_ccw,
             send_cw, recv_cw, send_ccw, recv_ccw,
             store_acc, ack_cw, ack_ccw):
        my = scal_ref[0]
        left = scal_ref[1]
        right = scal_ref[2]

        bar = pltpu.get_barrier_semaphore()
        pl.semaphore_signal(bar, inc=1, device_id=(left,), device_id_type=MESH)
        pl.semaphore_signal(bar, inc=1, device_id=(right,), device_id_type=MESH)
        pl.semaphore_wait(bar, 2)

        def partial_cw(idx):
            return jnp.dot(
                x_ref[pl.ds(idx * CH, CH), :], w_ref[:, :NH],
                preferred_element_type=jnp.float32,
                precision=lax.Precision.HIGHEST,
            )

        def partial_ccw(idx):
            return jnp.dot(
                x_ref[pl.ds(idx * CH, CH), :], w_ref[:, NH:],
                preferred_element_type=jnp.float32,
                precision=lax.Precision.HIGHEST,
            )

        def rs_desc(direction, slot, sub):
            cols = pl.ds(sub * HQ, HQ)
            if direction == 0:
                return pltpu.make_async_remote_copy(
                    src_ref=acc_cw.at[:, cols],
                    dst_ref=comm_cw.at[slot, :, cols],
                    send_sem=send_cw.at[NSUB * slot + sub],
                    recv_sem=recv_cw.at[NSUB * slot + sub],
                    device_id=(right,), device_id_type=MESH)
            return pltpu.make_async_remote_copy(
                src_ref=acc_ccw.at[:, cols],
                dst_ref=comm_ccw.at[slot, :, cols],
                send_sem=send_ccw.at[NSUB * slot + sub],
                recv_sem=recv_ccw.at[NSUB * slot + sub],
                device_id=(left,), device_id_type=MESH)

        def wait_ack_pair():
            pl.semaphore_wait(ack_cw, 1)
            pl.semaphore_wait(ack_ccw, 1)

        def signal_ack_pair():
            pl.semaphore_signal(ack_cw, inc=1, device_id=(left,),
                                device_id_type=MESH)
            pl.semaphore_signal(ack_ccw, inc=1, device_id=(right,),
                                device_id_type=MESH)

        acc_cw[...] = partial_cw(my)
        acc_ccw[...] = partial_ccw(my)
        for sub in range(NSUB):
            rs_desc(0, 0, sub).start()
            rs_desc(1, 0, sub).start()
        tmp_cw[...] = partial_cw((my - 1) % NDEV)
        tmp_ccw[...] = partial_ccw((my + 1) % NDEV)

        def rs_iter(k, slot, nslot, do_wait, last):
            for sub in range(NSUB):
                lo = sub * HQ
                cols = slice(lo, lo + HQ)
                rs_desc(0, slot, sub).wait()
                acc_cw[:, cols] = comm_cw[slot, :, cols] + tmp_cw[:, cols]
                if not last:
                    if do_wait and sub == 0:
                        pl.semaphore_wait(ack_cw, 1)
                    rs_desc(0, nslot, sub).start()
                rs_desc(1, slot, sub).wait()
                acc_ccw[:, cols] = comm_ccw[slot, :, cols] + tmp_ccw[:, cols]
                if not last:
                    if do_wait and sub == 0:
                        pl.semaphore_wait(ack_ccw, 1)
                    rs_desc(1, nslot, sub).start()
            if not last:
                tmp_cw[...] = partial_cw((my - k - 2) % NDEV)
                tmp_ccw[...] = partial_ccw((my + k + 2) % NDEV)
            signal_ack_pair()

        rs_iter(0, 0, 1, do_wait=False, last=False)

        def rs_pair(j, carry):
            k = 2 * j + 1
            rs_iter(k, 1, 0, do_wait=True, last=False)
            rs_iter(k + 1, 0, 1, do_wait=True, last=False)
            return carry

        lax.fori_loop(0, (NDEV - 4) // 2, rs_pair, 0)
        rs_iter(NDEV - 3, 1, 0, do_wait=True, last=False)
        rs_iter(NDEV - 2, 0, 1, do_wait=True, last=True)

        st1 = pltpu.make_async_copy(
            acc_cw,
            out_ref.at[pl.ds(((my + 1) % NDEV) * CH, CH), pl.ds(0, NH)],
            store_acc.at[0])
        st2 = pltpu.make_async_copy(
            acc_ccw,
            out_ref.at[pl.ds(((my - 1) % NDEV) * CH, CH), pl.ds(NH, NH)],
            store_acc.at[1])
        st1.start()
        st2.start()

        def ag_desc(direction, t_idx, slot, sub, from_acc):
            cols0 = sub * HQ if direction == 0 else NH + sub * HQ
            cols = pl.ds(cols0, HQ)
            if direction == 0:
                rows = pl.ds(((my + 1 - t_idx) % NDEV) * CH, CH)
                src = (acc_cw.at[:, pl.ds(sub * HQ, HQ)] if from_acc
                       else out_ref.at[rows, cols])
                return pltpu.make_async_remote_copy(
                    src_ref=src, dst_ref=out_ref.at[rows, cols],
                    send_sem=send_cw.at[NSUB * slot + sub],
                    recv_sem=recv_cw.at[NSUB * slot + sub],
                    device_id=(right,), device_id_type=MESH)
            rows = pl.ds(((my - 1 + t_idx) % NDEV) * CH, CH)
            src = (acc_ccw.at[:, pl.ds(sub * HQ, HQ)] if from_acc
                   else out_ref.at[rows, cols])
            return pltpu.make_async_remote_copy(
                src_ref=src, dst_ref=out_ref.at[rows, cols],
                send_sem=send_ccw.at[NSUB * slot + sub],
                recv_sem=recv_ccw.at[NSUB * slot + sub],
                device_id=(left,), device_id_type=MESH)

        pl.semaphore_wait(ack_cw, 1)
        pl.semaphore_wait(ack_ccw, 1)
        for sub in range(NSUB):
            ag_desc(0, 0, 1, sub, from_acc=True).start()
            ag_desc(1, 0, 1, sub, from_acc=True).start()

        def ag_iter(t, slot, nslot, last, first=False):
            for sub in range(NSUB):
                ag_desc(0, t, slot, sub, from_acc=first).wait()
                if not last:
                    if sub == 0:
                        pl.semaphore_wait(ack_cw, 1)
                    ag_desc(0, t + 1, nslot, sub, from_acc=False).start()
                ag_desc(1, t, slot, sub, from_acc=first).wait()
                if not last:
                    if sub == 0:
                        pl.semaphore_wait(ack_ccw, 1)
                    ag_desc(1, t + 1, nslot, sub, from_acc=False).start()
            signal_ack_pair()

        ag_iter(0, 1, 0, last=False, first=True)

        def ag_pair(j, carry):
            t = 2 * j + 1
            ag_iter(t, 0, 1, last=False)
            ag_iter(t + 1, 1, 0, last=False)
            return carry

        lax.fori_loop(0, (NDEV - 4) // 2, ag_pair, 0)
        ag_iter(NDEV - 3, 0, 1, last=False)
        ag_iter(NDEV - 2, 1, 0, last=True)

        st1.wait()
        st2.wait()
        pl.semaphore_wait(ack_cw, 2)
        pl.semaphore_wait(ack_ccw, 2)

    try:
        params = pltpu.CompilerParams(collective_id=0)
    except AttributeError:
        params = pltpu.TPUCompilerParams(collective_id=0)

    i = lax.axis_index("i")
    pos_t = jnp.asarray(POS_OF_LOGICAL, dtype=jnp.int32)
    ring_t = jnp.asarray(RING_LOGICAL, dtype=jnp.int32)
    r = pos_t[i]
    scalars = jnp.stack([
        r,
        ring_t[(r - 1) % NDEV],
        ring_t[(r + 1) % NDEV],
    ]).astype(jnp.int32)

    return pl.pallas_call(
        body,
        out_shape=jax.ShapeDtypeStruct((M, N), jnp.float32),
        in_specs=[
            pl.BlockSpec(memory_space=pltpu.MemorySpace.SMEM),
            pl.BlockSpec(memory_space=pltpu.VMEM),
            pl.BlockSpec(memory_space=pltpu.VMEM),
        ],
        out_specs=pl.BlockSpec(memory_space=pl.ANY),
        scratch_shapes=[
            pltpu.VMEM((CH, NH), jnp.float32),
            pltpu.VMEM((CH, NH), jnp.float32),
            pltpu.VMEM((CH, NH), jnp.float32),
            pltpu.VMEM((CH, NH), jnp.float32),
            pltpu.VMEM((2, CH, NH), jnp.float32),
            pltpu.VMEM((2, CH, NH), jnp.float32),
            pltpu.SemaphoreType.DMA((2 * NSUB,)),
            pltpu.SemaphoreType.DMA((2 * NSUB,)),
            pltpu.SemaphoreType.DMA((2 * NSUB,)),
            pltpu.SemaphoreType.DMA((2 * NSUB,)),
            pltpu.SemaphoreType.DMA((2,)),
            pltpu.SemaphoreType.REGULAR,
            pltpu.SemaphoreType.REGULAR,
        ],
        compiler_params=params,
    )(scalars, x, w_mat)


# device time: 1504570 ns/iter; 1.0002x vs baseline; 1.0002x over previous
import jax
import jax.numpy as jnp
from jax import lax
from jax.experimental import pallas as pl
from jax.experimental.pallas import tpu as pltpu

NDEV = 32
M = 8192
KSH = 256
N = 4096
NH = N // 2
HQ = NH // 2
CH = M // NDEV

MESH = pl.DeviceIdType.MESH

_PLANE_ORDER = [(0, 0), (1, 0), (1, 1), (0, 1),
                (0, 2), (1, 2), (1, 3), (0, 3)]
_LOGICAL_OF_COORD = {}
for _z in range(4):
    for _x, _y in _PLANE_ORDER:
        _LOGICAL_OF_COORD[(_x, _y, _z)] = len(_LOGICAL_OF_COORD)

_PATH_YZ = [(0, 0), (1, 0), (2, 0), (3, 0), (3, 1), (2, 1), (1, 1), (0, 1),
            (0, 2), (1, 2), (2, 2), (3, 2), (3, 3), (2, 3), (1, 3), (0, 3)]
_RING_COORDS = ([(0, y, z) for y, z in _PATH_YZ]
                + [(1, y, z) for y, z in reversed(_PATH_YZ)])
RING_LOGICAL = [_LOGICAL_OF_COORD[c] for c in _RING_COORDS]
POS_OF_LOGICAL = [0] * NDEV
for _p, _l in enumerate(RING_LOGICAL):
    POS_OF_LOGICAL[_l] = _p


def kernel(x, w_mat):
    def body(scal_ref, x_ref, w_ref, out_ref,
             acc_cw, acc_ccw, tmp_cw, tmp_ccw, comm_cw, comm_ccw,
             send_cw, recv_cw, send_ccw, recv_ccw,
             store_acc, ack_cw, ack_ccw):
        my = scal_ref[0]
        left = scal_ref[1]
        right = scal_ref[2]

        bar = pltpu.get_barrier_semaphore()
        pl.semaphore_signal(bar, inc=1, device_id=(left,), device_id_type=MESH)
        pl.semaphore_signal(bar, inc=1, device_id=(right,), device_id_type=MESH)
        pl.semaphore_wait(bar, 2)

        def partial_cw(idx):
            return jnp.dot(
                x_ref[pl.ds(idx * CH, CH), :], w_ref[:, :NH],
                preferred_element_type=jnp.float32,
                precision=lax.Precision.HIGHEST,
            )

        def partial_ccw(idx):
            return jnp.dot(
                x_ref[pl.ds(idx * CH, CH), :], w_ref[:, NH:],
                preferred_element_type=jnp.float32,
                precision=lax.Precision.HIGHEST,
            )

        def rs_desc(direction, slot, sub):
            cols = pl.ds(sub * HQ, HQ)
            if direction == 0:
                return pltpu.make_async_remote_copy(
                    src_ref=acc_cw.at[:, cols],
                    dst_ref=comm_cw.at[slot, :, cols],
                    send_sem=send_cw.at[2 * slot + sub],
                    recv_sem=recv_cw.at[2 * slot + sub],
                    device_id=(right,), device_id_type=MESH)
            return pltpu.make_async_remote_copy(
                src_ref=acc_ccw.at[:, cols],
                dst_ref=comm_ccw.at[slot, :, cols],
                send_sem=send_ccw.at[2 * slot + sub],
                recv_sem=recv_ccw.at[2 * slot + sub],
                device_id=(left,), device_id_type=MESH)

        def wait_ack_pair():
            pl.semaphore_wait(ack_cw, 1)
            pl.semaphore_wait(ack_ccw, 1)

        def signal_ack_pair():
            pl.semaphore_signal(ack_cw, inc=1, device_id=(left,),
                                device_id_type=MESH)
            pl.semaphore_signal(ack_ccw, inc=1, device_id=(right,),
                                device_id_type=MESH)

        acc_cw[...] = partial_cw(my)
        acc_ccw[...] = partial_ccw(my)
        for sub in (0, 1):
            rs_desc(0, 0, sub).start()
            rs_desc(1, 0, sub).start()
        tmp_cw[...] = partial_cw((my - 1) % NDEV)
        tmp_ccw[...] = partial_ccw((my + 1) % NDEV)

        def rs_iter(k, slot, nslot, do_wait, last):
            for sub in (0, 1):
                lo = sub * HQ
                cols = slice(lo, lo + HQ)
                rs_desc(0, slot, sub).wait()
                acc_cw[:, cols] = comm_cw[slot, :, cols] + tmp_cw[:, cols]
                if not last:
                    if do_wait and sub == 0:
                        pl.semaphore_wait(ack_cw, 1)
                    rs_desc(0, nslot, sub).start()
                rs_desc(1, slot, sub).wait()
                acc_ccw[:, cols] = comm_ccw[slot, :, cols] + tmp_ccw[:, cols]
                if not last:
                    if do_wait and sub == 0:
                        pl.semaphore_wait(ack_ccw, 1)
                    rs_desc(1, nslot, sub).start()
            if not last:
                tmp_cw[...] = partial_cw((my - k - 2) % NDEV)
                tmp_ccw[...] = partial_ccw((my + k + 2) % NDEV)
            signal_ack_pair()

        rs_iter(0, 0, 1, do_wait=False, last=False)

        def rs_pair(j, carry):
            k = 2 * j + 1
            rs_iter(k, 1, 0, do_wait=True, last=False)
            rs_iter(k + 1, 0, 1, do_wait=True, last=False)
            return carry

        lax.fori_loop(0, (NDEV - 4) // 2, rs_pair, 0)
        rs_iter(NDEV - 3, 1, 0, do_wait=True, last=False)
        rs_iter(NDEV - 2, 0, 1, do_wait=True, last=True)

        st1 = pltpu.make_async_copy(
            acc_cw,
            out_ref.at[pl.ds(((my + 1) % NDEV) * CH, CH), pl.ds(0, NH)],
            store_acc.at[0])
        st2 = pltpu.make_async_copy(
            acc_ccw,
            out_ref.at[pl.ds(((my - 1) % NDEV) * CH, CH), pl.ds(NH, NH)],
            store_acc.at[1])
        st1.start()
        st2.start()

        def ag_desc(direction, t_idx, slot, sub, from_acc):
            cols0 = sub * HQ if direction == 0 else NH + sub * HQ
            cols = pl.ds(cols0, HQ)
            if direction == 0:
                rows = pl.ds(((my + 1 - t_idx) % NDEV) * CH, CH)
                src = (acc_cw.at[:, pl.ds(sub * HQ, HQ)] if from_acc
                       else out_ref.at[rows, cols])
                return pltpu.make_async_remote_copy(
                    src_ref=src, dst_ref=out_ref.at[rows, cols],
                    send_sem=send_cw.at[2 * slot + sub],
                    recv_sem=recv_cw.at[2 * slot + sub],
                    device_id=(right,), device_id_type=MESH)
            rows = pl.ds(((my - 1 + t_idx) % NDEV) * CH, CH)
            src = (acc_ccw.at[:, pl.ds(sub * HQ, HQ)] if from_acc
                   else out_ref.at[rows, cols])
            return pltpu.make_async_remote_copy(
                src_ref=src, dst_ref=out_ref.at[rows, cols],
                send_sem=send_ccw.at[2 * slot + sub],
                recv_sem=recv_ccw.at[2 * slot + sub],
                device_id=(left,), device_id_type=MESH)

        pl.semaphore_wait(ack_cw, 1)
        pl.semaphore_wait(ack_ccw, 1)
        for sub in (0, 1):
            ag_desc(0, 0, 1, sub, from_acc=True).start()
            ag_desc(1, 0, 1, sub, from_acc=True).start()

        def ag_iter(t, slot, nslot, last, first=False):
            for sub in (0, 1):
                ag_desc(0, t, slot, sub, from_acc=first).wait()
                if not last:
                    if sub == 0:
                        pl.semaphore_wait(ack_cw, 1)
                    ag_desc(0, t + 1, nslot, sub, from_acc=False).start()
                ag_desc(1, t, slot, sub, from_acc=first).wait()
                if not last:
                    if sub == 0:
                        pl.semaphore_wait(ack_ccw, 1)
                    ag_desc(1, t + 1, nslot, sub, from_acc=False).start()
            signal_ack_pair()

        ag_iter(0, 1, 0, last=False, first=True)

        def ag_pair(j, carry):
            t = 2 * j + 1
            ag_iter(t, 0, 1, last=False)
            ag_iter(t + 1, 1, 0, last=False)
            return carry

        lax.fori_loop(0, (NDEV - 4) // 2, ag_pair, 0)
        ag_iter(NDEV - 3, 0, 1, last=False)
        ag_iter(NDEV - 2, 1, 0, last=True)

        st1.wait()
        st2.wait()
        pl.semaphore_wait(ack_cw, 2)
        pl.semaphore_wait(ack_ccw, 2)

    try:
        params = pltpu.CompilerParams(collective_id=0)
    except AttributeError:
        params = pltpu.TPUCompilerParams(collective_id=0)

    i = lax.axis_index("i")
    pos_t = jnp.asarray(POS_OF_LOGICAL, dtype=jnp.int32)
    ring_t = jnp.asarray(RING_LOGICAL, dtype=jnp.int32)
    r = pos_t[i]
    scalars = jnp.stack([
        r,
        ring_t[(r - 1) % NDEV],
        ring_t[(r + 1) % NDEV],
    ]).astype(jnp.int32)

    return pl.pallas_call(
        body,
        out_shape=jax.ShapeDtypeStruct((M, N), jnp.float32),
        in_specs=[
            pl.BlockSpec(memory_space=pltpu.MemorySpace.SMEM),
            pl.BlockSpec(memory_space=pltpu.VMEM),
            pl.BlockSpec(memory_space=pltpu.VMEM),
        ],
        out_specs=pl.BlockSpec(memory_space=pl.ANY),
        scratch_shapes=[
            pltpu.VMEM((CH, NH), jnp.float32),
            pltpu.VMEM((CH, NH), jnp.float32),
            pltpu.VMEM((CH, NH), jnp.float32),
            pltpu.VMEM((CH, NH), jnp.float32),
            pltpu.VMEM((2, CH, NH), jnp.float32),
            pltpu.VMEM((2, CH, NH), jnp.float32),
            pltpu.SemaphoreType.DMA((4,)),
            pltpu.SemaphoreType.DMA((4,)),
            pltpu.SemaphoreType.DMA((4,)),
            pltpu.SemaphoreType.DMA((4,)),
            pltpu.SemaphoreType.DMA((2,)),
            pltpu.SemaphoreType.REGULAR,
            pltpu.SemaphoreType.REGULAR,
        ],
        compiler_params=params,
    )(scalars, x, w_mat)
